# Initial kernel scaffold; baseline (speedup 1.0000x reference)
import jax
import jax.numpy as jnp
from jax.experimental import pallas as pl


def kernel(f, bases_c, bases_s, bases_0, directed_edges, node_weights, weights_c, weights_s, weights_0):
    raise NotImplementedError("write your pallas kernel here")


# speedup vs baseline: 6.0281x; 6.0281x over previous
import functools

import jax
import jax.numpy as jnp
from jax import lax
from jax.experimental import pallas as pl
from jax.experimental.pallas import tpu as pltpu
from jax.experimental.pallas import tpu_sc as plsc

_NC = 2
_NS = 16
_CH = 32
_LN = 16

_GDN = lax.GatherDimensionNumbers(
    offset_dims=(), collapsed_slice_dims=(0,), start_index_map=(0,))


def _bcast_lane(vec, lane):
    idx = (jnp.zeros((_LN,), jnp.int32) + lane)[:, None]
    return lax.gather(vec, idx, _GDN, (1,),
                      mode=lax.GatherScatterMode.PROMISE_IN_BOUNDS)


def _v_table_body(f_ref, w_ref, bc_ref, bs_ref, v_ref, *, o, k):
    g = lax.dot_general(f_ref[...], w_ref[...],
                        (((0,), (0,)), ((), ())),
                        preferred_element_type=jnp.float32)
    v_ref[:, :o] = g[:, :o]
    for kk in range(k):
        gc = g[:, o * (1 + kk):o * (2 + kk)]
        gs = g[:, o * (1 + k + kk):o * (2 + k + kk)]
        bck = bc_ref[:, kk:kk + 1]
        bsk = bs_ref[:, kk:kk + 1]
        v_ref[:, o * (1 + kk):o * (2 + kk)] = 2.0 * (bck * gc + bsk * gs)
        v_ref[:, o * (1 + k + kk):o * (2 + k + kk)] = 2.0 * (bsk * gc - bck * gs)


def _make_sc_edge_kernel(n, e, o, vw):
    nw_workers = _NC * _NS
    tch = e // _CH
    nit = (tch + nw_workers - 1) // nw_workers
    rpt = ((n // _NS) // 8 + 1) * 8
    rlast = n - (_NS - 1) * rpt
    zc = 8
    mesh = plsc.VectorSubcoreMesh(core_axis_name="c", subcore_axis_name="s")

    @functools.partial(
        pl.kernel,
        out_type=jax.ShapeDtypeStruct((_NC, n, o), jnp.float32),
        mesh=mesh,
        scratch_types=[
            pltpu.VMEM((_CH,), jnp.int32),
            pltpu.VMEM((_CH,), jnp.int32),
            pltpu.VMEM((_CH + _LN,), jnp.float32),
            pltpu.VMEM((_CH, o), jnp.float32),
            pltpu.VMEM((_CH, vw), jnp.float32),
            pltpu.VMEM_SHARED((n, o), jnp.float32),
            pltpu.SemaphoreType.DMA,
            pltpu.SemaphoreType.DMA,
        ],
    )
    def sc_edges(v_hbm, co_hbm, src_hbm, tgt_hbm, nw_hbm, out_hbm,
                 idx_s, idx_t, nwb, cmsg, vrows, acc, sem1, sem2):
        c = lax.axis_index("c")
        s = lax.axis_index("s")
        wid = s * _NC + c
        my_rows = jnp.where(s == _NS - 1, rlast, rpt)
        my_base = s * rpt

        def zrow(r, carry):
            for j in range(o // _LN):
                cmsg[r, pl.ds(j * _LN, _LN)] = jnp.zeros((_LN,), jnp.float32)
            return carry
        lax.fori_loop(0, zc, zrow, 0)

        def zcopy(t, carry):
            pltpu.sync_copy(cmsg.at[pl.ds(0, zc)],
                            acc.at[pl.ds(my_base + t * zc, zc)])
            return carry
        lax.fori_loop(0, my_rows // zc, zcopy, 0)
        plsc.subcore_barrier()

        def chunk_body(ci, carry):
            cid = wid + ci * nw_workers

            @pl.when(cid < tch)
            def _do_chunk():
                base = cid * _CH
                pltpu.sync_copy(src_hbm.at[pl.ds(base, _CH)], idx_s)
                pltpu.sync_copy(tgt_hbm.at[pl.ds(base, _CH)], idx_t)
                pltpu.sync_copy(nw_hbm.at[pl.ds(base, _CH)],
                                nwb.at[pl.ds(0, _CH)])
                pltpu.async_copy(v_hbm.at[idx_s], vrows, sem1).wait()
                pltpu.async_copy(co_hbm.at[idx_t], cmsg, sem2).wait()

                def edge_body(ei, ecarry):
                    grp = (ei // _LN) * _LN
                    nwv = _bcast_lane(nwb[pl.ds(grp, _LN)], ei - grp)
                    srow = cmsg[ei, pl.ds(0, _LN)] * nwv
                    cf = [_bcast_lane(srow, kk) for kk in range(8)]
                    for j in range(o // _LN):
                        a = nwv * vrows[ei, pl.ds(j * _LN, _LN)]
                        for kk in range(4):
                            a = a + cf[kk] * vrows[
                                ei, pl.ds((1 + kk) * o + j * _LN, _LN)]
                            a = a + cf[4 + kk] * vrows[
                                ei, pl.ds((5 + kk) * o + j * _LN, _LN)]
                        cmsg[ei, pl.ds(j * _LN, _LN)] = a
                    return ecarry
                lax.fori_loop(0, _CH, edge_body, 0)

                pltpu.sync_copy(cmsg, acc.at[idx_t], add=True)
            return carry
        lax.fori_loop(0, nit, chunk_body, 0)
        plsc.subcore_barrier()

        def ocopy(t, carry):
            rows = pl.ds(my_base + t * zc, zc)
            pltpu.sync_copy(acc.at[rows], out_hbm.at[c, rows])
            return carry
        lax.fori_loop(0, my_rows // zc, ocopy, 0)

    return sc_edges


def kernel(f, bases_c, bases_s, bases_0, directed_edges, node_weights,
           weights_c, weights_s, weights_0):
    del bases_0
    b, i, n = f.shape
    o = weights_0.shape[1]
    k = weights_c.shape[2]
    e = directed_edges.shape[1]
    vw = (2 * k + 1) * o

    fp = f[0]
    bc = bases_c[0, :, :, 0]
    bs = bases_s[0, :, :, 0]
    tgt = directed_edges[0, :, 0, 0]
    src = directed_edges[0, :, 1, 0]
    nw = node_weights[0, :, 0]
    w0 = weights_0[:, :, 0, 0]
    wcat = jnp.concatenate(
        [w0] + [weights_c[:, :, kk, 0] for kk in range(k)]
             + [weights_s[:, :, kk, 0] for kk in range(k)], axis=1)

    nb = 1024
    npad = ((n + nb - 1) // nb) * nb
    fpad = jnp.pad(fp, ((0, 0), (0, npad - n)))
    bcp = jnp.pad(bc, ((0, npad - n), (0, 0)))
    bsp = jnp.pad(bs, ((0, npad - n), (0, 0)))

    v_table = pl.pallas_call(
        functools.partial(_v_table_body, o=o, k=k),
        grid=(npad // nb,),
        in_specs=[
            pl.BlockSpec((i, nb), lambda g: (0, g)),
            pl.BlockSpec((i, vw), lambda g: (0, 0)),
            pl.BlockSpec((nb, k), lambda g: (g, 0)),
            pl.BlockSpec((nb, k), lambda g: (g, 0)),
        ],
        out_specs=pl.BlockSpec((nb, vw), lambda g: (g, 0)),
        out_shape=jax.ShapeDtypeStruct((npad, vw), jnp.float32),
    )(fpad, wcat, bcp, bsp)

    coeff = jnp.concatenate(
        [bc, bs, jnp.zeros((n, o - 2 * k), jnp.float32)], axis=1)

    halves = _make_sc_edge_kernel(n, e, o, vw)(v_table, coeff, src, tgt, nw)
    out = halves[0] + halves[1]
    return jnp.transpose(out)[None]


# speedup vs baseline: 7.7924x; 1.2927x over previous
import functools

import numpy as np

import jax
import jax.numpy as jnp
from jax import lax
from jax.experimental import pallas as pl
from jax.experimental.pallas import tpu as pltpu
from jax.experimental.pallas import tpu_sc as plsc

_NC = 2
_NS = 16
_CH = 32
_LN = 16

_GDN = lax.GatherDimensionNumbers(
    offset_dims=(), collapsed_slice_dims=(0,), start_index_map=(0,))


def _bcast_lane(vec, lane):
    idx = (jnp.zeros((_LN,), jnp.int32) + lane)[:, None]
    return lax.gather(vec, idx, _GDN, (1,),
                      mode=lax.GatherScatterMode.PROMISE_IN_BOUNDS)


def _pack_rows(piece, o):
    words = []
    for g in range(o // 32):
        lo = lax.bitcast_convert_type(
            piece[:, g * 32:g * 32 + 16].astype(jnp.bfloat16),
            jnp.uint16).astype(jnp.int32)
        hi = lax.bitcast_convert_type(
            piece[:, g * 32 + 16:g * 32 + 32].astype(jnp.bfloat16),
            jnp.uint16).astype(jnp.int32)
        words.append(lo | (hi << 16))
    return jnp.concatenate(words, axis=1)


def _v_table_body(f_ref, w_ref, bc_ref, bs_ref, v_ref, *, o, k, tw):
    g = lax.dot_general(f_ref[...], w_ref[...],
                        (((0,), (0,)), ((), ())),
                        preferred_element_type=jnp.float32)
    ow = o // 2
    v_ref[:, :ow] = _pack_rows(g[:, :o], o)
    for kk in range(k):
        gc = g[:, o * (1 + kk):o * (2 + kk)]
        gs = g[:, o * (1 + k + kk):o * (2 + k + kk)]
        bck = bc_ref[:, kk:kk + 1]
        bsk = bs_ref[:, kk:kk + 1]
        v_ref[:, ow * (1 + kk):ow * (2 + kk)] = _pack_rows(
            2.0 * (bck * gc + bsk * gs), o)
        v_ref[:, ow * (1 + k + kk):ow * (2 + k + kk)] = _pack_rows(
            2.0 * (bsk * gc - bck * gs), o)
    v_ref[:, ow * (2 * k + 1):] = jnp.zeros(
        (v_ref.shape[0], tw - ow * (2 * k + 1)), jnp.int32)


def _make_sc_edge_kernel(n, e, o, tw):
    nw_workers = _NC * _NS
    tch = e // _CH
    nit = (tch + nw_workers - 1) // nw_workers
    rpt = ((n // _NS) // 8 + 1) * 8
    rlast = n - (_NS - 1) * rpt
    zc = 8
    mesh = plsc.VectorSubcoreMesh(core_axis_name="c", subcore_axis_name="s")

    @functools.partial(
        pl.kernel,
        out_type=jax.ShapeDtypeStruct((_NC, n, o), jnp.float32),
        mesh=mesh,
        scratch_types=[
            pltpu.VMEM((_CH,), jnp.int32),
            pltpu.VMEM((_CH,), jnp.int32),
            pltpu.VMEM((_CH + _LN,), jnp.float32),
            pltpu.VMEM((_CH, o), jnp.float32),
            pltpu.VMEM((_CH, tw), jnp.int32),
            pltpu.VMEM_SHARED((n, o), jnp.float32),
            pltpu.SemaphoreType.DMA,
            pltpu.SemaphoreType.DMA,
        ],
    )
    def sc_edges(v_hbm, co_hbm, src_hbm, tgt_hbm, nw_hbm, out_hbm,
                 idx_s, idx_t, nwb, cmsg, vrows, acc, sem1, sem2):
        c = lax.axis_index("c")
        s = lax.axis_index("s")
        wid = s * _NC + c
        my_rows = jnp.where(s == _NS - 1, rlast, rpt)
        my_base = s * rpt

        def zrow(r, carry):
            for j in range(o // _LN):
                cmsg[r, pl.ds(j * _LN, _LN)] = jnp.zeros((_LN,), jnp.float32)
            return carry
        lax.fori_loop(0, zc, zrow, 0)

        def zcopy(t, carry):
            pltpu.sync_copy(cmsg.at[pl.ds(0, zc)],
                            acc.at[pl.ds(my_base + t * zc, zc)])
            return carry
        lax.fori_loop(0, my_rows // zc, zcopy, 0)
        plsc.subcore_barrier()

        def chunk_body(ci, carry):
            cid = wid + ci * nw_workers

            @pl.when(cid < tch)
            def _do_chunk():
                base = cid * _CH
                pltpu.sync_copy(src_hbm.at[pl.ds(base, _CH)], idx_s)
                pltpu.sync_copy(tgt_hbm.at[pl.ds(base, _CH)], idx_t)
                pltpu.sync_copy(nw_hbm.at[pl.ds(base, _CH)],
                                nwb.at[pl.ds(0, _CH)])
                pltpu.async_copy(v_hbm.at[idx_s], vrows, sem1).wait()
                pltpu.async_copy(co_hbm.at[idx_t], cmsg, sem2).wait()

                def edge_body(ei, ecarry):
                    grp = (ei // _LN) * _LN
                    nwv = _bcast_lane(nwb[pl.ds(grp, _LN)], ei - grp)
                    srow = cmsg[ei, pl.ds(0, _LN)] * nwv
                    cf = [_bcast_lane(srow, kk) for kk in range(8)]

                    def vld2(term, j):
                        w = vrows[ei, pl.ds(term * (o // 2) + j * _LN, _LN)]
                        lo = lax.bitcast_convert_type(w << 16, jnp.float32)
                        hi = lax.bitcast_convert_type(
                            w & jnp.int32(-65536), jnp.float32)
                        return lo, hi

                    for j in range(o // (2 * _LN)):
                        alo, ahi = vld2(0, j)
                        alo = nwv * alo
                        ahi = nwv * ahi
                        for kk in range(4):
                            plo, phi = vld2(1 + kk, j)
                            alo = alo + cf[kk] * plo
                            ahi = ahi + cf[kk] * phi
                            qlo, qhi = vld2(5 + kk, j)
                            alo = alo + cf[4 + kk] * qlo
                            ahi = ahi + cf[4 + kk] * qhi
                        cmsg[ei, pl.ds(j * 2 * _LN, _LN)] = alo
                        cmsg[ei, pl.ds(j * 2 * _LN + _LN, _LN)] = ahi
                    return ecarry
                lax.fori_loop(0, _CH, edge_body, 0)

                pltpu.sync_copy(cmsg, acc.at[idx_t], add=True)
            return carry
        lax.fori_loop(0, nit, chunk_body, 0)
        plsc.subcore_barrier()

        def ocopy(t, carry):
            rows = pl.ds(my_base + t * zc, zc)
            pltpu.sync_copy(acc.at[rows], out_hbm.at[c, rows])
            return carry
        lax.fori_loop(0, my_rows // zc, ocopy, 0)

    return sc_edges


def kernel(f, bases_c, bases_s, bases_0, directed_edges, node_weights,
           weights_c, weights_s, weights_0):
    del bases_0
    b, i, n = f.shape
    o = weights_0.shape[1]
    k = weights_c.shape[2]
    e = directed_edges.shape[1]
    vw = (2 * k + 1) * o

    fp = f[0]
    bc = bases_c[0, :, :, 0]
    bs = bases_s[0, :, :, 0]
    tgt = directed_edges[0, :, 0, 0]
    src = directed_edges[0, :, 1, 0]
    nw = node_weights[0, :, 0]
    w0 = weights_0[:, :, 0, 0]
    wcat = jnp.concatenate(
        [w0] + [weights_c[:, :, kk, 0] for kk in range(k)]
             + [weights_s[:, :, kk, 0] for kk in range(k)], axis=1)
    tw = ((vw // 2 + 127) // 128) * 128

    nb = 1024
    npad = ((n + nb - 1) // nb) * nb
    fpad = jnp.pad(fp, ((0, 0), (0, npad - n)))
    bcp = jnp.pad(bc, ((0, npad - n), (0, 0)))
    bsp = jnp.pad(bs, ((0, npad - n), (0, 0)))

    v_table = pl.pallas_call(
        functools.partial(_v_table_body, o=o, k=k, tw=tw),
        grid=(npad // nb,),
        in_specs=[
            pl.BlockSpec((i, nb), lambda g: (0, g)),
            pl.BlockSpec((i, vw), lambda g: (0, 0)),
            pl.BlockSpec((nb, k), lambda g: (g, 0)),
            pl.BlockSpec((nb, k), lambda g: (g, 0)),
        ],
        out_specs=pl.BlockSpec((nb, tw), lambda g: (g, 0)),
        out_shape=jax.ShapeDtypeStruct((npad, tw), jnp.int32),
    )(fpad, wcat, bcp, bsp)

    coeff = jnp.concatenate(
        [bc, bs, jnp.zeros((n, o - 2 * k), jnp.float32)], axis=1)

    halves = _make_sc_edge_kernel(n, e, o, tw)(v_table, coeff, src, tgt, nw)
    out = halves[0] + halves[1]
    return jnp.transpose(out)[None]


# speedup vs baseline: 13.5736x; 1.7419x over previous
import functools

import numpy as np

import jax
import jax.numpy as jnp
from jax import lax
from jax.experimental import pallas as pl
from jax.experimental.pallas import tpu as pltpu
from jax.experimental.pallas import tpu_sc as plsc

_NC = 2
_NS = 16
_CH = 32
_LN = 16

_GDN = lax.GatherDimensionNumbers(
    offset_dims=(), collapsed_slice_dims=(0,), start_index_map=(0,))


def _bcast_lane(vec, lane):
    idx = (jnp.zeros((_LN,), jnp.int32) + lane)[:, None]
    return lax.gather(vec, idx, _GDN, (1,),
                      mode=lax.GatherScatterMode.PROMISE_IN_BOUNDS)


def _pack_rows(piece, o):
    words = []
    for g in range(o // 32):
        lo = lax.bitcast_convert_type(
            piece[:, g * 32:g * 32 + 16].astype(jnp.bfloat16),
            jnp.uint16).astype(jnp.int32)
        hi = lax.bitcast_convert_type(
            piece[:, g * 32 + 16:g * 32 + 32].astype(jnp.bfloat16),
            jnp.uint16).astype(jnp.int32)
        words.append(lo | (hi << 16))
    return jnp.concatenate(words, axis=1)


def _v_table_body(f_ref, w_ref, bc_ref, bs_ref, v_ref, *, o, k, tw):
    g = lax.dot_general(f_ref[...], w_ref[...],
                        (((0,), (0,)), ((), ())),
                        preferred_element_type=jnp.float32)
    ow = o // 2
    v_ref[:, :ow] = _pack_rows(g[:, :o], o)
    for kk in range(k):
        gc = g[:, o * (1 + kk):o * (2 + kk)]
        gs = g[:, o * (1 + k + kk):o * (2 + k + kk)]
        bck = bc_ref[:, kk:kk + 1]
        bsk = bs_ref[:, kk:kk + 1]
        v_ref[:, ow * (1 + kk):ow * (2 + kk)] = _pack_rows(
            2.0 * (bck * gc + bsk * gs), o)
        v_ref[:, ow * (1 + k + kk):ow * (2 + k + kk)] = _pack_rows(
            2.0 * (bsk * gc - bck * gs), o)
    v_ref[:, ow * (2 * k + 1):] = jnp.zeros(
        (v_ref.shape[0], tw - ow * (2 * k + 1)), jnp.int32)


def _make_sc_edge_kernel(n, e, o, tw):
    nw_workers = _NC * _NS
    tch = e // _CH
    nit = (tch + nw_workers - 1) // nw_workers
    rpt = ((n // _NS) // 8 + 1) * 8
    rlast = n - (_NS - 1) * rpt
    zc = 8
    mesh = plsc.VectorSubcoreMesh(core_axis_name="c", subcore_axis_name="s")

    @functools.partial(
        pl.kernel,
        out_type=jax.ShapeDtypeStruct((_NC, n, o), jnp.float32),
        mesh=mesh,
        scratch_types=[
            pltpu.VMEM((_CH,), jnp.int32),
            pltpu.VMEM((_CH,), jnp.int32),
            pltpu.VMEM((_CH,), jnp.int32),
            pltpu.VMEM((_CH,), jnp.int32),
            pltpu.VMEM((_CH + _LN,), jnp.float32),
            pltpu.VMEM((_CH + _LN,), jnp.float32),
            pltpu.VMEM((_CH, o), jnp.float32),
            pltpu.VMEM((_CH, o), jnp.float32),
            pltpu.VMEM((_CH, tw), jnp.int32),
            pltpu.VMEM((_CH, tw), jnp.int32),
            pltpu.VMEM_SHARED((n, o), jnp.float32),
            pltpu.SemaphoreType.DMA,
            pltpu.SemaphoreType.DMA,
            pltpu.SemaphoreType.DMA,
            pltpu.SemaphoreType.DMA,
            pltpu.SemaphoreType.DMA,
            pltpu.SemaphoreType.DMA,
        ],
    )
    def sc_edges(v_hbm, co_hbm, src_hbm, tgt_hbm, nw_hbm, out_hbm,
                 idx_s0, idx_s1, idx_t0, idx_t1, nwb0, nwb1, cmsg0, cmsg1,
                 vrows0, vrows1, acc,
                 isem0, isem1, vsem0, vsem1, csem0, csem1):
        slots = ((idx_s0, idx_t0, nwb0, cmsg0, vrows0, isem0, vsem0, csem0),
                 (idx_s1, idx_t1, nwb1, cmsg1, vrows1, isem1, vsem1, csem1))
        c = lax.axis_index("c")
        s = lax.axis_index("s")
        wid = s * _NC + c
        my_rows = jnp.where(s == _NS - 1, rlast, rpt)
        my_base = s * rpt

        def zrow(r, carry):
            for j in range(o // _LN):
                cmsg0[r, pl.ds(j * _LN, _LN)] = jnp.zeros((_LN,), jnp.float32)
            return carry
        lax.fori_loop(0, zc, zrow, 0)

        def zcopy(t, carry):
            pltpu.sync_copy(cmsg0.at[pl.ds(0, zc)],
                            acc.at[pl.ds(my_base + t * zc, zc)])
            return carry
        lax.fori_loop(0, my_rows // zc, zcopy, 0)
        plsc.subcore_barrier()

        def stage1(cid, sl):
            idx_s, idx_t, nwb, _, _, isem, _, _ = sl
            base = cid * _CH
            pltpu.async_copy(src_hbm.at[pl.ds(base, _CH)], idx_s, isem)
            pltpu.async_copy(tgt_hbm.at[pl.ds(base, _CH)], idx_t, isem)
            pltpu.async_copy(nw_hbm.at[pl.ds(base, _CH)],
                             nwb.at[pl.ds(0, _CH)], isem)

        def stage2(sl):
            idx_s, idx_t, nwb, cmsg, vrows, isem, vsem, csem = sl
            pltpu.make_async_copy(
                src_hbm.at[pl.ds(0, _CH)], idx_s, isem).wait()
            pltpu.make_async_copy(
                tgt_hbm.at[pl.ds(0, _CH)], idx_t, isem).wait()
            pltpu.make_async_copy(
                nw_hbm.at[pl.ds(0, _CH)], nwb.at[pl.ds(0, _CH)], isem).wait()
            pltpu.async_copy(v_hbm.at[idx_s], vrows, vsem)
            pltpu.async_copy(co_hbm.at[idx_t], cmsg, csem)

        def gather_wait(sl):
            idx_s, idx_t, nwb, cmsg, vrows, isem, vsem, csem = sl
            pltpu.make_async_copy(v_hbm.at[idx_s], vrows, vsem).wait()
            pltpu.make_async_copy(co_hbm.at[idx_t], cmsg, csem).wait()

        def compute_chunk(sl):
            idx_s, idx_t, nwb, cmsg, vrows, isem, vsem, csem = sl

            def edge_body(ei, ecarry):
                grp = (ei // _LN) * _LN
                nwv = _bcast_lane(nwb[pl.ds(grp, _LN)], ei - grp)
                srow = cmsg[ei, pl.ds(0, _LN)] * nwv
                cf = [_bcast_lane(srow, kk) for kk in range(8)]

                def vld2(term, j):
                    w = vrows[ei, pl.ds(term * (o // 2) + j * _LN, _LN)]
                    lo = lax.bitcast_convert_type(w << 16, jnp.float32)
                    hi = lax.bitcast_convert_type(
                        w & jnp.int32(-65536), jnp.float32)
                    return lo, hi

                for j in range(o // (2 * _LN)):
                    alo, ahi = vld2(0, j)
                    alo = nwv * alo
                    ahi = nwv * ahi
                    for kk in range(4):
                        plo, phi = vld2(1 + kk, j)
                        alo = alo + cf[kk] * plo
                        ahi = ahi + cf[kk] * phi
                        qlo, qhi = vld2(5 + kk, j)
                        alo = alo + cf[4 + kk] * qlo
                        ahi = ahi + cf[4 + kk] * qhi
                    cmsg[ei, pl.ds(j * 2 * _LN, _LN)] = alo
                    cmsg[ei, pl.ds(j * 2 * _LN + _LN, _LN)] = ahi
                return ecarry
            lax.fori_loop(0, _CH, edge_body, 0)

        stage1(wid, slots[0])
        stage1(wid + nw_workers, slots[1])
        stage2(slots[0])

        def pair_body(it, carry):
            for b in (0, 1):
                ci = 2 * it + b
                cid = wid + ci * nw_workers

                @pl.when(cid < tch)
                def _do_chunk():
                    sl = slots[b]
                    gather_wait(sl)

                    @pl.when(cid + nw_workers < tch)
                    def _fire_next_gathers():
                        stage2(slots[1 - b])

                    compute_chunk(sl)
                    pltpu.sync_copy(sl[3], acc.at[sl[1]], add=True)

                    @pl.when(cid + 2 * nw_workers < tch)
                    def _prefetch_ids():
                        stage1(cid + 2 * nw_workers, sl)
            return carry
        lax.fori_loop(0, (nit + 1) // 2, pair_body, 0)
        plsc.subcore_barrier()

        def ocopy(t, carry):
            rows = pl.ds(my_base + t * zc, zc)
            pltpu.sync_copy(acc.at[rows], out_hbm.at[c, rows])
            return carry
        lax.fori_loop(0, my_rows // zc, ocopy, 0)

    return sc_edges


def kernel(f, bases_c, bases_s, bases_0, directed_edges, node_weights,
           weights_c, weights_s, weights_0):
    del bases_0
    b, i, n = f.shape
    o = weights_0.shape[1]
    k = weights_c.shape[2]
    e = directed_edges.shape[1]
    vw = (2 * k + 1) * o

    fp = f[0]
    bc = bases_c[0, :, :, 0]
    bs = bases_s[0, :, :, 0]
    tgt = directed_edges[0, :, 0, 0]
    src = directed_edges[0, :, 1, 0]
    nw = node_weights[0, :, 0]
    w0 = weights_0[:, :, 0, 0]
    wcat = jnp.concatenate(
        [w0] + [weights_c[:, :, kk, 0] for kk in range(k)]
             + [weights_s[:, :, kk, 0] for kk in range(k)], axis=1)
    tw = ((vw // 2 + 127) // 128) * 128

    nb = 1024
    npad = ((n + nb - 1) // nb) * nb
    fpad = jnp.pad(fp, ((0, 0), (0, npad - n)))
    bcp = jnp.pad(bc, ((0, npad - n), (0, 0)))
    bsp = jnp.pad(bs, ((0, npad - n), (0, 0)))

    v_table = pl.pallas_call(
        functools.partial(_v_table_body, o=o, k=k, tw=tw),
        grid=(npad // nb,),
        in_specs=[
            pl.BlockSpec((i, nb), lambda g: (0, g)),
            pl.BlockSpec((i, vw), lambda g: (0, 0)),
            pl.BlockSpec((nb, k), lambda g: (g, 0)),
            pl.BlockSpec((nb, k), lambda g: (g, 0)),
        ],
        out_specs=pl.BlockSpec((nb, tw), lambda g: (g, 0)),
        out_shape=jax.ShapeDtypeStruct((npad, tw), jnp.int32),
    )(fpad, wcat, bcp, bsp)

    coeff = jnp.concatenate(
        [bc, bs, jnp.zeros((n, o - 2 * k), jnp.float32)], axis=1)

    halves = _make_sc_edge_kernel(n, e, o, tw)(v_table, coeff, src, tgt, nw)
    out = halves[0] + halves[1]
    return jnp.transpose(out)[None]


# speedup vs baseline: 14.2971x; 1.0533x over previous
import functools

import numpy as np

import jax
import jax.numpy as jnp
from jax import lax
from jax.experimental import pallas as pl
from jax.experimental.pallas import tpu as pltpu
from jax.experimental.pallas import tpu_sc as plsc

_NC = 2
_NS = 16
_CH = 32
_LN = 16

_GDN = lax.GatherDimensionNumbers(
    offset_dims=(), collapsed_slice_dims=(0,), start_index_map=(0,))


def _bcast_lane(vec, lane):
    idx = (jnp.zeros((_LN,), jnp.int32) + lane)[:, None]
    return lax.gather(vec, idx, _GDN, (1,),
                      mode=lax.GatherScatterMode.PROMISE_IN_BOUNDS)


def _pack_rows(piece, o):
    words = []
    for g in range(o // 32):
        lo = lax.bitcast_convert_type(
            piece[:, g * 32:g * 32 + 16].astype(jnp.bfloat16),
            jnp.uint16).astype(jnp.int32)
        hi = lax.bitcast_convert_type(
            piece[:, g * 32 + 16:g * 32 + 32].astype(jnp.bfloat16),
            jnp.uint16).astype(jnp.int32)
        words.append(lo | (hi << 16))
    return jnp.concatenate(words, axis=1)


def _v_table_body(f_ref, w_ref, bc_ref, bs_ref, v_ref, *, o, k, tw):
    g = lax.dot_general(f_ref[...], w_ref[...],
                        (((0,), (0,)), ((), ())),
                        preferred_element_type=jnp.float32)
    ow = o // 2
    v_ref[:, :ow] = _pack_rows(g[:, :o], o)
    for kk in range(k):
        gc = g[:, o * (1 + kk):o * (2 + kk)]
        gs = g[:, o * (1 + k + kk):o * (2 + k + kk)]
        bck = bc_ref[:, kk:kk + 1]
        bsk = bs_ref[:, kk:kk + 1]
        v_ref[:, ow * (1 + kk):ow * (2 + kk)] = _pack_rows(
            2.0 * (bck * gc + bsk * gs), o)
        v_ref[:, ow * (1 + k + kk):ow * (2 + k + kk)] = _pack_rows(
            2.0 * (bsk * gc - bck * gs), o)
    v_ref[:, ow * (2 * k + 1):] = jnp.zeros(
        (v_ref.shape[0], tw - ow * (2 * k + 1)), jnp.int32)


def _make_sc_edge_kernel(n, e, o, tw):
    nw_workers = _NC * _NS
    tch = e // _CH
    nit = (tch + nw_workers - 1) // nw_workers
    rpt = ((n // _NS) // 8 + 1) * 8
    rlast = n - (_NS - 1) * rpt
    zc = 8
    mesh = plsc.VectorSubcoreMesh(core_axis_name="c", subcore_axis_name="s")

    @functools.partial(
        pl.kernel,
        out_type=jax.ShapeDtypeStruct((_NC, n, o), jnp.float32),
        mesh=mesh,
        scratch_types=[
            pltpu.VMEM((_CH,), jnp.int32),
            pltpu.VMEM((_CH,), jnp.int32),
            pltpu.VMEM((_CH,), jnp.int32),
            pltpu.VMEM((_CH,), jnp.int32),
            pltpu.VMEM((_CH + _LN,), jnp.float32),
            pltpu.VMEM((_CH + _LN,), jnp.float32),
            pltpu.VMEM((_CH, o), jnp.float32),
            pltpu.VMEM((_CH, o), jnp.float32),
            pltpu.VMEM((_CH, tw), jnp.int32),
            pltpu.VMEM((_CH, tw), jnp.int32),
            pltpu.VMEM_SHARED((n, o), jnp.float32),
            pltpu.SemaphoreType.DMA,
            pltpu.SemaphoreType.DMA,
            pltpu.SemaphoreType.DMA,
            pltpu.SemaphoreType.DMA,
            pltpu.SemaphoreType.DMA,
            pltpu.SemaphoreType.DMA,
            pltpu.SemaphoreType.DMA,
            pltpu.SemaphoreType.DMA,
        ],
    )
    def sc_edges(v_hbm, co_hbm, src_hbm, tgt_hbm, nw_hbm, out_hbm,
                 idx_s0, idx_s1, idx_t0, idx_t1, nwb0, nwb1, cmsg0, cmsg1,
                 vrows0, vrows1, acc,
                 isem0, isem1, vsem0, vsem1, csem0, csem1, ssem0, ssem1):
        slots = ((idx_s0, idx_t0, nwb0, cmsg0, vrows0, isem0, vsem0, csem0,
                  ssem0),
                 (idx_s1, idx_t1, nwb1, cmsg1, vrows1, isem1, vsem1, csem1,
                  ssem1))
        c = lax.axis_index("c")
        s = lax.axis_index("s")
        wid = s * _NC + c
        my_rows = jnp.where(s == _NS - 1, rlast, rpt)
        my_base = s * rpt

        def zrow(r, carry):
            for j in range(o // _LN):
                cmsg0[r, pl.ds(j * _LN, _LN)] = jnp.zeros((_LN,), jnp.float32)
            return carry
        lax.fori_loop(0, zc, zrow, 0)

        def zcopy(t, carry):
            pltpu.sync_copy(cmsg0.at[pl.ds(0, zc)],
                            acc.at[pl.ds(my_base + t * zc, zc)])
            return carry
        lax.fori_loop(0, my_rows // zc, zcopy, 0)
        plsc.subcore_barrier()

        def stage1(cid, sl):
            idx_s, idx_t, nwb, _, _, isem, _, _, _ = sl
            base = cid * _CH
            pltpu.async_copy(src_hbm.at[pl.ds(base, _CH)], idx_s, isem)
            pltpu.async_copy(tgt_hbm.at[pl.ds(base, _CH)], idx_t, isem)
            pltpu.async_copy(nw_hbm.at[pl.ds(base, _CH)],
                             nwb.at[pl.ds(0, _CH)], isem)

        def stage2(sl, drain_scatter):
            idx_s, idx_t, nwb, cmsg, vrows, isem, vsem, csem, ssem = sl
            if drain_scatter is not None:
                @pl.when(drain_scatter)
                def _drain():
                    pltpu.make_async_copy(cmsg, acc.at[idx_t], ssem).wait()
            pltpu.make_async_copy(
                src_hbm.at[pl.ds(0, _CH)], idx_s, isem).wait()
            pltpu.make_async_copy(
                tgt_hbm.at[pl.ds(0, _CH)], idx_t, isem).wait()
            pltpu.make_async_copy(
                nw_hbm.at[pl.ds(0, _CH)], nwb.at[pl.ds(0, _CH)], isem).wait()
            pltpu.async_copy(v_hbm.at[idx_s], vrows, vsem)
            pltpu.async_copy(co_hbm.at[idx_t], cmsg, csem)

        def gather_wait(sl):
            idx_s, idx_t, nwb, cmsg, vrows, isem, vsem, csem, ssem = sl
            pltpu.make_async_copy(v_hbm.at[idx_s], vrows, vsem).wait()
            pltpu.make_async_copy(co_hbm.at[idx_t], cmsg, csem).wait()

        def compute_chunk(sl):
            idx_s, idx_t, nwb, cmsg, vrows, isem, vsem, csem, ssem = sl

            def edge_body(ei, ecarry):
                grp = (ei // _LN) * _LN
                nwv = _bcast_lane(nwb[pl.ds(grp, _LN)], ei - grp)
                srow = cmsg[ei, pl.ds(0, _LN)] * nwv
                cf = [_bcast_lane(srow, kk) for kk in range(8)]

                def vld2(term, j):
                    w = vrows[ei, pl.ds(term * (o // 2) + j * _LN, _LN)]
                    lo = lax.bitcast_convert_type(w << 16, jnp.float32)
                    hi = lax.bitcast_convert_type(
                        w & jnp.int32(-65536), jnp.float32)
                    return lo, hi

                for j in range(o // (2 * _LN)):
                    alo, ahi = vld2(0, j)
                    alo = nwv * alo
                    ahi = nwv * ahi
                    for kk in range(4):
                        plo, phi = vld2(1 + kk, j)
                        alo = alo + cf[kk] * plo
                        ahi = ahi + cf[kk] * phi
                        qlo, qhi = vld2(5 + kk, j)
                        alo = alo + cf[4 + kk] * qlo
                        ahi = ahi + cf[4 + kk] * qhi
                    cmsg[ei, pl.ds(j * 2 * _LN, _LN)] = alo
                    cmsg[ei, pl.ds(j * 2 * _LN + _LN, _LN)] = ahi
                return ecarry
            lax.fori_loop(0, _CH, edge_body, 0)

        stage1(wid, slots[0])
        stage1(wid + nw_workers, slots[1])
        stage2(slots[0], None)

        def pair_body(it, carry):
            for b in (0, 1):
                ci = 2 * it + b
                cid = wid + ci * nw_workers

                @pl.when(cid < tch)
                def _do_chunk():
                    sl = slots[b]
                    gather_wait(sl)

                    @pl.when(cid + nw_workers < tch)
                    def _fire_next_gathers():
                        stage2(slots[1 - b], ci >= 1 if b == 0 else True)

                    compute_chunk(sl)
                    pltpu.async_copy(sl[3], acc.at[sl[1]], sl[8], add=True)

                    @pl.when(cid + 2 * nw_workers < tch)
                    def _prefetch_ids():
                        stage1(cid + 2 * nw_workers, sl)
            return carry
        lax.fori_loop(0, (nit + 1) // 2, pair_body, 0)
        for b in (0, 1):
            pltpu.make_async_copy(slots[b][3], acc.at[slots[b][1]],
                                  slots[b][8]).wait()
        plsc.subcore_barrier()

        def ocopy(t, carry):
            rows = pl.ds(my_base + t * zc, zc)
            pltpu.sync_copy(acc.at[rows], out_hbm.at[c, rows])
            return carry
        lax.fori_loop(0, my_rows // zc, ocopy, 0)

    return sc_edges


def kernel(f, bases_c, bases_s, bases_0, directed_edges, node_weights,
           weights_c, weights_s, weights_0):
    del bases_0
    b, i, n = f.shape
    o = weights_0.shape[1]
    k = weights_c.shape[2]
    e = directed_edges.shape[1]
    vw = (2 * k + 1) * o

    fp = f[0]
    bc = bases_c[0, :, :, 0]
    bs = bases_s[0, :, :, 0]
    tgt = directed_edges[0, :, 0, 0]
    src = directed_edges[0, :, 1, 0]
    nw = node_weights[0, :, 0]
    w0 = weights_0[:, :, 0, 0]
    wcat = jnp.concatenate(
        [w0] + [weights_c[:, :, kk, 0] for kk in range(k)]
             + [weights_s[:, :, kk, 0] for kk in range(k)], axis=1)
    tw = ((vw // 2 + 127) // 128) * 128

    nb = 1024
    npad = ((n + nb - 1) // nb) * nb
    fpad = jnp.pad(fp, ((0, 0), (0, npad - n)))
    bcp = jnp.pad(bc, ((0, npad - n), (0, 0)))
    bsp = jnp.pad(bs, ((0, npad - n), (0, 0)))

    v_table = pl.pallas_call(
        functools.partial(_v_table_body, o=o, k=k, tw=tw),
        grid=(npad // nb,),
        in_specs=[
            pl.BlockSpec((i, nb), lambda g: (0, g)),
            pl.BlockSpec((i, vw), lambda g: (0, 0)),
            pl.BlockSpec((nb, k), lambda g: (g, 0)),
            pl.BlockSpec((nb, k), lambda g: (g, 0)),
        ],
        out_specs=pl.BlockSpec((nb, tw), lambda g: (g, 0)),
        out_shape=jax.ShapeDtypeStruct((npad, tw), jnp.int32),
    )(fpad, wcat, bcp, bsp)

    coeff = jnp.concatenate(
        [bc, bs, jnp.zeros((n, o - 2 * k), jnp.float32)], axis=1)

    halves = _make_sc_edge_kernel(n, e, o, tw)(v_table, coeff, src, tgt, nw)
    out = halves[0] + halves[1]
    return jnp.transpose(out)[None]
